# baseline (speedup 1.0000x reference)
import jax
import jax.numpy as jnp
from jax.experimental import pallas as pl

B, T, C, S = 16, 512, 2048, 64
S2 = 2 * S


def _emis_kernel(llh_ref, s2pn_ref, s2pd_ref, out_ref):
    llh = llh_ref[0]
    s2p = jnp.concatenate([s2pn_ref[0], s2pd_ref[...]], axis=-1)
    cidx = jax.lax.broadcasted_iota(jnp.int32, (C, S2), 0)
    onehot = (cidx == s2p).astype(jnp.float32)
    out_ref[...] = jnp.dot(llh, onehot, preferred_element_type=jnp.float32)


def _fwd_kernel(emis_ref, nAt_ref, dAt_ref, nI_ref, dI_ref, nF_ref, dF_ref,
                seql_ref, out_ref):
    Wn = jnp.exp(nAt_ref[...]).reshape(S, B * S)
    Wnh = Wn.astype(jnp.bfloat16)
    Wnl = (Wn - Wnh.astype(jnp.float32)).astype(jnp.bfloat16)
    Wd = jnp.exp(dAt_ref[...]).reshape(S, S)
    Wdh = Wd.astype(jnp.bfloat16)
    Wdl = (Wd - Wdh.astype(jnp.float32)).astype(jnp.bfloat16)
    zb_n = jnp.zeros((S, B * S), jnp.bfloat16)
    zb_d = jnp.zeros((S, 2 * S), jnp.bfloat16)
    pad_d = jnp.zeros((S, S), jnp.bfloat16)
    r0 = jnp.concatenate([Wnh, zb_d], axis=1)
    r1 = jnp.concatenate([zb_n, Wdh, pad_d], axis=1)
    r2 = jnp.concatenate([Wnh, zb_d], axis=1)
    r3 = jnp.concatenate([zb_n, Wdh, pad_d], axis=1)
    r4 = jnp.concatenate([Wnl, zb_d], axis=1)
    r5 = jnp.concatenate([zb_n, Wdl, pad_d], axis=1)
    WS = jnp.concatenate([r0, r1, r2, r3, r4, r5], axis=0)
    pairM = (jax.lax.broadcasted_iota(jnp.int32, (B, B // 2, S2), 1) ==
             jax.lax.broadcasted_iota(jnp.int32, (B, B // 2, S2), 0) // 2
             ).astype(jnp.float32)
    oddM = (jax.lax.broadcasted_iota(jnp.int32, (B, S2), 0) % 2) == 1
    seql = seql_ref[...]
    e0 = emis_ref[0]
    a0n = nI_ref[...] + e0[:, :S]
    a0d = jnp.broadcast_to(dI_ref[...], (B, S)) + e0[:, S:]
    on = jnp.max(a0n, axis=1, keepdims=True)
    od = jnp.max(a0d, axis=1, keepdims=True)
    vn = jnp.exp(a0n - on)
    vd = jnp.exp(a0d - od)
    Rn = jnp.ones((B, 1), jnp.float32)
    Rd = jnp.ones((B, 1), jnp.float32)
    Ln = jnp.zeros((B, 1), jnp.float32)
    Ld = jnp.zeros((B, 1), jnp.float32)

    def step2(t, carry):
        vn, vd, on, od, Rn, Rd, Ln, Ld = carry
        ge = jnp.exp(emis_ref[t])
        gn = ge[:, :S] * Rn
        gd = ge[:, S:] * Rd
        vnh = vn.astype(jnp.bfloat16)
        vnl = (vn - vnh.astype(jnp.float32)).astype(jnp.bfloat16)
        vdh = vd.astype(jnp.bfloat16)
        vdl = (vd - vdh.astype(jnp.float32)).astype(jnp.bfloat16)
        vS = jnp.concatenate([vnh, vdh, vnl, vdl, vnh, vdh], axis=1)
        sc = jnp.dot(vS, WS, preferred_element_type=jnp.float32)
        ud = sc[:, B * S:B * S + S]
        s10 = jnp.sum(
            sc[:, :B * S].reshape(B, B // 2, S2) * pairM, axis=1)
        rolled = jnp.concatenate([s10[:, S:], s10[:, :S]], axis=1)
        un = jnp.where(oddM, rolled, s10)[:, :S]
        act = t < seql
        vn2 = jnp.where(act, un * gn, vn)
        vd2 = jnp.where(act, ud * gd, vd)
        on2 = jnp.where(act, on + Ln, on)
        od2 = jnp.where(act, od + Ld, od)
        Mn = jnp.max(vn2, axis=1, keepdims=True)
        Md = jnp.max(vd2, axis=1, keepdims=True)
        return (vn2, vd2, on2, od2, 1.0 / Mn, 1.0 / Md,
                jnp.log(Mn), jnp.log(Md))

    vn, vd, on, od, Rn, Rd, Ln, Ld = jax.lax.fori_loop(
        1, T, step2, (vn, vd, on, od, Rn, Rd, Ln, Ld))

    nfs = jnp.sum(vn * jnp.exp(nF_ref[...]), axis=1, keepdims=True)
    dfs = jnp.sum(vd * jnp.exp(jnp.broadcast_to(dF_ref[...], (B, S))),
                  axis=1, keepdims=True)
    num = on + jnp.log(nfs)
    den = od + jnp.log(dfs)
    out_ref[...] = -jnp.sum(num - den, axis=0, keepdims=True)


def _impl(input, seqlengths, num_logA, num_init, num_final, num_state2pdf,
          den_logA, den_init, den_final, den_state2pdf, interpret=False):
    emis = pl.pallas_call(
        _emis_kernel,
        grid=(B,),
        in_specs=[
            pl.BlockSpec((1, T, C), lambda b: (b, 0, 0)),
            pl.BlockSpec((1, 1, S), lambda b: (b, 0, 0)),
            pl.BlockSpec((1, S), lambda b: (0, 0)),
        ],
        out_specs=pl.BlockSpec((T, S2), lambda b: (0, b)),
        out_shape=jax.ShapeDtypeStruct((T, B * S2), jnp.float32),
        interpret=interpret,
    )(input, num_state2pdf.reshape(B, 1, S), den_state2pdf.reshape(1, S))
    loss = pl.pallas_call(
        _fwd_kernel,
        out_shape=jax.ShapeDtypeStruct((1, 1), jnp.float32),
        interpret=interpret,
    )(emis.reshape(T, B, S2), jnp.transpose(num_logA, (1, 0, 2)),
      den_logA.reshape(S, 1, S), num_init, den_init.reshape(1, S),
      num_final, den_final.reshape(1, S), seqlengths.reshape(B, 1))
    return loss[0, 0]


def kernel(input, seqlengths, num_logA, num_init, num_final, num_state2pdf,
           den_logA, den_init, den_final, den_state2pdf):
    return _impl(input, seqlengths, num_logA, num_init, num_final,
                 num_state2pdf, den_logA, den_init, den_final, den_state2pdf)


# speedup vs baseline: 1.1958x; 1.1958x over previous
import jax
import jax.numpy as jnp
from jax.experimental import pallas as pl

B, T, C, S = 16, 512, 2048, 64
S2 = 2 * S


def _emis_kernel(llh_ref, s2pn_ref, s2pd_ref, out_ref):
    llh = llh_ref[0]
    s2p = jnp.concatenate([s2pn_ref[0], s2pd_ref[...]], axis=-1)
    cidx = jax.lax.broadcasted_iota(jnp.int32, (C, S2), 0)
    onehot = (cidx == s2p).astype(jnp.float32)
    out_ref[...] = jnp.dot(llh, onehot, preferred_element_type=jnp.float32)


def _fwd_kernel(emis_ref, nAt_ref, dAt_ref, nI_ref, dI_ref, nF_ref, dF_ref,
                seql_ref, out_ref):
    Wn = jnp.exp(nAt_ref[...]).reshape(S, B * S)
    Wnh = Wn.astype(jnp.bfloat16)
    Wnl = (Wn - Wnh.astype(jnp.float32)).astype(jnp.bfloat16)
    Wd = jnp.exp(dAt_ref[...]).reshape(S, S)
    Wdh = Wd.astype(jnp.bfloat16)
    Wdl = (Wd - Wdh.astype(jnp.float32)).astype(jnp.bfloat16)
    WnS = jnp.concatenate([Wnh, Wnh, Wnl], axis=0)
    WdS = jnp.concatenate([Wdh, Wdh, Wdl], axis=0)
    pairM = (jax.lax.broadcasted_iota(jnp.int32, (B, B // 2, S2), 1) ==
             jax.lax.broadcasted_iota(jnp.int32, (B, B // 2, S2), 0) // 2
             ).astype(jnp.float32)
    oddM = (jax.lax.broadcasted_iota(jnp.int32, (B, S2), 0) % 2) == 1
    seql = seql_ref[...]
    e0 = emis_ref[0]
    a0n = nI_ref[...] + e0[:, :S]
    a0d = jnp.broadcast_to(dI_ref[...], (B, S)) + e0[:, S:]
    on = jnp.max(a0n, axis=1, keepdims=True)
    od = jnp.max(a0d, axis=1, keepdims=True)
    vn = jnp.exp(a0n - on)
    vd = jnp.exp(a0d - od)
    Rn = jnp.ones((B, 1), jnp.float32)
    Rd = jnp.ones((B, 1), jnp.float32)
    Ln = jnp.zeros((B, 1), jnp.float32)
    Ld = jnp.zeros((B, 1), jnp.float32)

    def step2(t, carry):
        vn, vd, on, od, Rn, Rd, Ln, Ld = carry
        ge = jnp.exp(emis_ref[t])
        gn = ge[:, :S] * Rn
        gd = ge[:, S:] * Rd
        vnh = vn.astype(jnp.bfloat16)
        vnl = (vn - vnh.astype(jnp.float32)).astype(jnp.bfloat16)
        vdh = vd.astype(jnp.bfloat16)
        vdl = (vd - vdh.astype(jnp.float32)).astype(jnp.bfloat16)
        vnS = jnp.concatenate([vnh, vnl, vnh], axis=1)
        vdS = jnp.concatenate([vdh, vdl, vdh], axis=1)
        scn = jnp.dot(vnS, WnS, preferred_element_type=jnp.float32)
        ud = jnp.dot(vdS, WdS, preferred_element_type=jnp.float32)
        s10 = jnp.sum(scn.reshape(B, B // 2, S2) * pairM, axis=1)
        rolled = jnp.concatenate([s10[:, S:], s10[:, :S]], axis=1)
        un = jnp.where(oddM, rolled, s10)[:, :S]
        act = t < seql
        vn2 = jnp.where(act, un * gn, vn)
        vd2 = jnp.where(act, ud * gd, vd)
        on2 = jnp.where(act, on + Ln, on)
        od2 = jnp.where(act, od + Ld, od)
        Mn = jnp.max(vn2, axis=1, keepdims=True)
        Md = jnp.max(vd2, axis=1, keepdims=True)
        return (vn2, vd2, on2, od2, 1.0 / Mn, 1.0 / Md,
                jnp.log(Mn), jnp.log(Md))

    def two_steps(k, carry):
        return step2(2 * k + 2, step2(2 * k + 1, carry))

    carry = jax.lax.fori_loop(
        0, (T - 2) // 2, two_steps, (vn, vd, on, od, Rn, Rd, Ln, Ld))
    vn, vd, on, od, Rn, Rd, Ln, Ld = step2(T - 1, carry)

    nfs = jnp.sum(vn * jnp.exp(nF_ref[...]), axis=1, keepdims=True)
    dfs = jnp.sum(vd * jnp.exp(jnp.broadcast_to(dF_ref[...], (B, S))),
                  axis=1, keepdims=True)
    num = on + jnp.log(nfs)
    den = od + jnp.log(dfs)
    out_ref[...] = -jnp.sum(num - den, axis=0, keepdims=True)


def _impl(input, seqlengths, num_logA, num_init, num_final, num_state2pdf,
          den_logA, den_init, den_final, den_state2pdf, interpret=False):
    emis = pl.pallas_call(
        _emis_kernel,
        grid=(B,),
        in_specs=[
            pl.BlockSpec((1, T, C), lambda b: (b, 0, 0)),
            pl.BlockSpec((1, 1, S), lambda b: (b, 0, 0)),
            pl.BlockSpec((1, S), lambda b: (0, 0)),
        ],
        out_specs=pl.BlockSpec((T, S2), lambda b: (0, b)),
        out_shape=jax.ShapeDtypeStruct((T, B * S2), jnp.float32),
        interpret=interpret,
    )(input, num_state2pdf.reshape(B, 1, S), den_state2pdf.reshape(1, S))
    loss = pl.pallas_call(
        _fwd_kernel,
        out_shape=jax.ShapeDtypeStruct((1, 1), jnp.float32),
        interpret=interpret,
    )(emis.reshape(T, B, S2), jnp.transpose(num_logA, (1, 0, 2)),
      den_logA.reshape(S, 1, S), num_init, den_init.reshape(1, S),
      num_final, den_final.reshape(1, S), seqlengths.reshape(B, 1))
    return loss[0, 0]


def kernel(input, seqlengths, num_logA, num_init, num_final, num_state2pdf,
           den_logA, den_init, den_final, den_state2pdf):
    return _impl(input, seqlengths, num_logA, num_init, num_final,
                 num_state2pdf, den_logA, den_init, den_final, den_state2pdf)


# speedup vs baseline: 1.2627x; 1.0559x over previous
import jax
import jax.numpy as jnp
from jax.experimental import pallas as pl

B, T, C, S = 16, 512, 2048, 64
S2 = 2 * S


def _emis_kernel(llh_ref, s2pn_ref, s2pd_ref, out_ref):
    llh = llh_ref[0]
    s2p = jnp.concatenate([s2pn_ref[0], s2pd_ref[...]], axis=-1)
    cidx = jax.lax.broadcasted_iota(jnp.int32, (C, S2), 0)
    onehot = (cidx == s2p).astype(jnp.float32)
    out_ref[...] = jnp.dot(llh, onehot, preferred_element_type=jnp.float32)


def _fwd_kernel(emis_ref, nAt_ref, dAt_ref, nI_ref, dI_ref, nF_ref, dF_ref,
                seql_ref, out_ref):
    Wn = jnp.exp(nAt_ref[...]).reshape(S, B * S)
    Wnh = Wn.astype(jnp.bfloat16)
    Wnl = (Wn - Wnh.astype(jnp.float32)).astype(jnp.bfloat16)
    Wd = jnp.exp(dAt_ref[...]).reshape(S, S)
    Wdh = Wd.astype(jnp.bfloat16)
    Wdl = (Wd - Wdh.astype(jnp.float32)).astype(jnp.bfloat16)
    WnS = jnp.concatenate([Wnh, Wnh, Wnl], axis=0)
    WdS = jnp.concatenate([Wdh, Wdh, Wdl], axis=0)
    pairM = (jax.lax.broadcasted_iota(jnp.int32, (B, B // 2, S2), 1) ==
             jax.lax.broadcasted_iota(jnp.int32, (B, B // 2, S2), 0) // 2
             ).astype(jnp.float32)
    oddM = (jax.lax.broadcasted_iota(jnp.int32, (B, S2), 0) % 2) == 1
    seql = seql_ref[...]
    e0 = emis_ref[0]
    a0n = nI_ref[...] + e0[:, :S]
    a0d = jnp.broadcast_to(dI_ref[...], (B, S)) + e0[:, S:]
    on = jnp.max(a0n, axis=1, keepdims=True)
    od = jnp.max(a0d, axis=1, keepdims=True)
    vn = jnp.exp(a0n - on)
    vd = jnp.exp(a0d - od)
    Rn = jnp.ones((B, 1), jnp.float32)
    Rd = jnp.ones((B, 1), jnp.float32)
    Ln = jnp.zeros((B, 1), jnp.float32)
    Ld = jnp.zeros((B, 1), jnp.float32)

    def step2(t, carry):
        vn, vd, on, od, Rn, Rd, Ln, Ld = carry
        ge = jnp.exp(emis_ref[t])
        gn = ge[:, :S] * Rn
        gd = ge[:, S:] * Rd
        vnh = vn.astype(jnp.bfloat16)
        vnl = (vn - vnh.astype(jnp.float32)).astype(jnp.bfloat16)
        vdh = vd.astype(jnp.bfloat16)
        vdl = (vd - vdh.astype(jnp.float32)).astype(jnp.bfloat16)
        vnS = jnp.concatenate([vnh, vnl, vnh], axis=1)
        vdS = jnp.concatenate([vdh, vdl, vdh], axis=1)
        scn = jnp.dot(vnS, WnS, preferred_element_type=jnp.float32)
        ud = jnp.dot(vdS, WdS, preferred_element_type=jnp.float32)
        s10 = jnp.sum(scn.reshape(B, B // 2, S2) * pairM, axis=1)
        rolled = jnp.concatenate([s10[:, S:], s10[:, :S]], axis=1)
        un = jnp.where(oddM, rolled, s10)[:, :S]
        act = t < seql
        vn2 = jnp.where(act, un * gn, vn)
        vd2 = jnp.where(act, ud * gd, vd)
        on2 = jnp.where(act, on + Ln, on)
        od2 = jnp.where(act, od + Ld, od)
        Mn = jnp.max(vn2, axis=1, keepdims=True)
        Md = jnp.max(vd2, axis=1, keepdims=True)
        return (vn2, vd2, on2, od2, 1.0 / Mn, 1.0 / Md,
                jnp.log(Mn), jnp.log(Md))

    def four_steps(k, carry):
        for i in range(4):
            carry = step2(4 * k + 1 + i, carry)
        return carry

    carry = (vn, vd, on, od, Rn, Rd, Ln, Ld)
    carry = jax.lax.fori_loop(0, (T - 4) // 4, four_steps, carry)
    for t in range(T - 3, T):
        carry = step2(t, carry)
    vn, vd, on, od, Rn, Rd, Ln, Ld = carry

    nfs = jnp.sum(vn * jnp.exp(nF_ref[...]), axis=1, keepdims=True)
    dfs = jnp.sum(vd * jnp.exp(jnp.broadcast_to(dF_ref[...], (B, S))),
                  axis=1, keepdims=True)
    num = on + jnp.log(nfs)
    den = od + jnp.log(dfs)
    out_ref[...] = -jnp.sum(num - den, axis=0, keepdims=True)


def _impl(input, seqlengths, num_logA, num_init, num_final, num_state2pdf,
          den_logA, den_init, den_final, den_state2pdf, interpret=False):
    emis = pl.pallas_call(
        _emis_kernel,
        grid=(B,),
        in_specs=[
            pl.BlockSpec((1, T, C), lambda b: (b, 0, 0)),
            pl.BlockSpec((1, 1, S), lambda b: (b, 0, 0)),
            pl.BlockSpec((1, S), lambda b: (0, 0)),
        ],
        out_specs=pl.BlockSpec((T, S2), lambda b: (0, b)),
        out_shape=jax.ShapeDtypeStruct((T, B * S2), jnp.float32),
        interpret=interpret,
    )(input, num_state2pdf.reshape(B, 1, S), den_state2pdf.reshape(1, S))
    loss = pl.pallas_call(
        _fwd_kernel,
        out_shape=jax.ShapeDtypeStruct((1, 1), jnp.float32),
        interpret=interpret,
    )(emis.reshape(T, B, S2), jnp.transpose(num_logA, (1, 0, 2)),
      den_logA.reshape(S, 1, S), num_init, den_init.reshape(1, S),
      num_final, den_final.reshape(1, S), seqlengths.reshape(B, 1))
    return loss[0, 0]


def kernel(input, seqlengths, num_logA, num_init, num_final, num_state2pdf,
           den_logA, den_init, den_final, den_state2pdf):
    return _impl(input, seqlengths, num_logA, num_init, num_final,
                 num_state2pdf, den_logA, den_init, den_final, den_state2pdf)
